# Initial kernel scaffold; baseline (speedup 1.0000x reference)
import jax
import jax.numpy as jnp
from jax.experimental import pallas as pl


def kernel(x_snp, x_gene, edge_sg, edge_gs, edge_gg, params):
    raise NotImplementedError("write your pallas kernel here")


# speedup vs baseline: 19.4364x; 19.4364x over previous
import functools

import jax
import jax.numpy as jnp
from jax import lax
from jax.experimental import pallas as pl
from jax.experimental.pallas import tpu as pltpu
from jax.experimental.pallas import tpu_sc as plsc

N = 50000
E = 200000
HID = 128
HEADS = 8
DHEAD = 16
OUT = 64
NSUB = 16
LANES = 16

f32 = jnp.float32
i32 = jnp.int32


def _build_sc_msg(n_nodes, n_edges, nchunk, ch, chp, eb, mb, interpret=False,
                  stage=99):
    kpc = nchunk // 2
    pt = n_edges // NSUB
    cnt = -((-(pt + 4)) // eb) * eb
    cap = pt + 2 * mb
    stripe = chp // NSUB
    zb = 80 if stripe % 80 == 0 else (40 if stripe % 40 == 0 else stripe)
    assert chp % NSUB == 0 and stripe % zb == 0 and eb % 16 == 0 and mb % 16 == 0
    nzb = stripe // zb
    dump = ch

    def body(src_hbm, dst_hbm, asrc_hbm, adst_hbm, x_hbm, out_hbm,
             den_sh, agg_sh,
             csrc, cgd, srcb, dstb,
             asr, adr, exr, denr, wbuf, xr, sidx,
             zbufa, zbufb, bounce, sem1, sem2, sem3):
        c = lax.axis_index("c")
        s = lax.axis_index("s")
        z16 = jnp.zeros((16,), f32)

        def zfa(i, _):
            for t in range(8):
                zbufa[i, pl.ds(16 * t, 16)] = z16
            return 0
        lax.fori_loop(0, zb, zfa, 0)

        def zfb(i, _):
            zbufb[i, :] = z16
            return 0
        lax.fori_loop(0, stripe, zfb, 0)

        base = s * stripe
        lane = lax.iota(i32, 16)

        for k in range(kpc):
            chunk = kpc * c + k
            lo = chunk * ch
            hi = lo + ch

            for t in range(nzb):
                pltpu.sync_copy(zbufa, agg_sh.at[pl.ds(base + zb * t, zb)])
            pltpu.sync_copy(zbufb, den_sh.at[pl.ds(base, stripe)])
            plsc.subcore_barrier()

            e0 = s * pt
            start = e0 - (e0 % 8)

            if stage < 2:
                continue

            def scan_batch(b, pos):
                ebase = pl.multiple_of(start + b * eb, 8)
                pltpu.sync_copy(src_hbm.at[pl.ds(ebase, eb)], srcb)
                pltpu.sync_copy(dst_hbm.at[pl.ds(ebase, eb)], dstb)

                def gat(vec, idxv):
                    return vec.at[idxv].get(mode="promise_in_bounds")

                def grp(g, pos):
                    sv = srcb[pl.ds(g * 16, 16)]
                    dv = dstb[pl.ds(g * 16, 16)]
                    eid = ebase + g * 16 + lane
                    m = ((eid >= e0) & (eid < e0 + pt)
                         & (dv >= lo) & (dv < hi))
                    mi = jnp.where(m, 1, 0)
                    x = mi
                    for sb in (1, 2, 4, 8):
                        sh = gat(x, jnp.maximum(lane - sb, 0))
                        x = x + jnp.where(lane >= sb, sh, 0)
                    tot = x[15]
                    r = jnp.where(m, lane - (x - mi), 0)
                    vi = mi
                    cx, dx = sv, dv
                    for sb in (1, 2, 4, 8):
                        si = jnp.minimum(lane + sb, 15)
                        cxs = gat(cx, si)
                        dxs = gat(dx, si)
                        rs = gat(r, si)
                        vs = gat(vi, si)
                        take = (lane + sb <= 15) & (vs > 0) & ((rs & sb) != 0)
                        keep = (vi > 0) & ((r & sb) == 0)
                        cx = jnp.where(take, cxs, cx)
                        dx = jnp.where(take, dxs, dx)
                        r = jnp.where(take, rs - sb, r)
                        vi = jnp.where(take, 1, jnp.where(keep, 1, 0))
                    csrc[pl.ds(pos, 16)] = cx
                    cgd[pl.ds(pos, 16)] = dx
                    return pos + tot

                return lax.fori_loop(0, eb // 16, grp, pos)

            pos = lax.fori_loop(0, cnt // eb, scan_batch, 0)

            if stage < 3 or 20 <= stage < 30:
                continue

            gpad = jnp.full((16,), lo + dump, i32)
            zpad = jnp.zeros((16,), i32)
            for t in range(mb // 16):
                csrc[pl.ds(pos + 16 * t, 16)] = zpad
                cgd[pl.ds(pos + 16 * t, 16)] = gpad
            nbat = (pos + mb - 1) // mb

            def fill_sidx(off):
                def cp(j, _):
                    sidx[pl.ds(j * 16, 16)] = cgd[pl.ds(off + j * 16, 16)] - lo
                    return 0
                lax.fori_loop(0, mb // 16, cp, 0)

            def logits(off):
                cp1 = pltpu.async_copy(
                    asrc_hbm.at[csrc.at[pl.ds(off, mb)]], asr, sem1)
                cp2 = pltpu.async_copy(
                    adst_hbm.at[cgd.at[pl.ds(off, mb)]], adr, sem2)
                cp1.wait()
                cp2.wait()

            def den_b(b, _):
                off = b * mb
                logits(off)

                def exrow(j, _):
                    av = asr[j, :] + adr[j, :]
                    av = jnp.where(av > 0, av, 0.2 * av)
                    exr[j, :] = jnp.exp(av)
                    return 0
                lax.fori_loop(0, mb, exrow, 0)
                fill_sidx(off)
                pltpu.sync_copy(exr, den_sh.at[sidx], add=True)
                return 0
            lax.fori_loop(0, nbat, den_b, 0)
            plsc.subcore_barrier()

            if stage < 4:
                continue

            def msg_b(b, _):
                off = b * mb
                cp3 = pltpu.async_copy(
                    x_hbm.at[csrc.at[pl.ds(off, mb)]], xr, sem3)
                fill_sidx(off)
                pltpu.sync_copy(den_sh.at[sidx], denr)
                logits(off)

                def wrow(j, _):
                    av = asr[j, :] + adr[j, :]
                    av = jnp.where(av > 0, av, 0.2 * av)
                    wbuf[j, :] = jnp.exp(av) / (denr[j, :] + 1e-16)
                    return 0
                lax.fori_loop(0, mb, wrow, 0)
                cp3.wait()

                def scale(j, _):
                    wv = wbuf[j, :]
                    for h in range(HEADS):
                        xr[j, pl.ds(h * 16, 16)] = (
                            xr[j, pl.ds(h * 16, 16)] * wv[h])
                    return 0
                lax.fori_loop(0, mb, scale, 0)
                pltpu.sync_copy(xr, agg_sh.at[sidx], add=True)
                return 0
            lax.fori_loop(0, nbat, msg_b, 0)
            plsc.subcore_barrier()

            for t in range(nzb):
                pltpu.sync_copy(agg_sh.at[pl.ds(base + zb * t, zb)], bounce)
                pltpu.sync_copy(bounce,
                                out_hbm.at[chunk, pl.ds(base + zb * t, zb)])
            plsc.subcore_barrier()

    mesh = plsc.VectorSubcoreMesh(core_axis_name="c", subcore_axis_name="s",
                                  num_cores=2, num_subcores=NSUB)
    fn = pl.kernel(
        body,
        out_type=jax.ShapeDtypeStruct((nchunk, chp, HID), f32),
        mesh=mesh,
        scratch_types=dict(
            den_sh=pltpu.VMEM_SHARED((chp, 16), f32),
            agg_sh=pltpu.VMEM_SHARED((chp, HID), f32),
            csrc=pltpu.VMEM((cap,), i32),
            cgd=pltpu.VMEM((cap,), i32),
            srcb=pltpu.VMEM((eb,), i32),
            dstb=pltpu.VMEM((eb,), i32),
            asr=pltpu.VMEM((mb, 16), f32),
            adr=pltpu.VMEM((mb, 16), f32),
            exr=pltpu.VMEM((mb, 16), f32),
            denr=pltpu.VMEM((mb, 16), f32),
            wbuf=pltpu.VMEM((mb, 16), f32),
            xr=pltpu.VMEM((mb, HID), f32),
            sidx=pltpu.VMEM((mb,), i32),
            zbufa=pltpu.VMEM((zb, HID), f32),
            zbufb=pltpu.VMEM((stripe, 16), f32),
            bounce=pltpu.VMEM((zb, HID), f32),
            sem1=pltpu.SemaphoreType.DMA,
            sem2=pltpu.SemaphoreType.DMA,
            sem3=pltpu.SemaphoreType.DMA,
        ),
        compiler_params=pltpu.CompilerParams(use_tc_tiling_on_sc=False),
        interpret=interpret,
    )
    return fn


_NCHUNK = 16
_CH = N // _NCHUNK
_CHP = 3200


@functools.cache
def _sc_msg_fn():
    return _build_sc_msg(N, E, _NCHUNK, _CH, _CHP, 512, 128)


_EDGE_PAD = (15 * (E // NSUB) - 4 + 12800) - E


def _sc_messages(src, dst, a_src, a_dst, x_src):
    src_p = jnp.pad(src, (0, _EDGE_PAD))
    dst_p = jnp.pad(dst, (0, _EDGE_PAD))
    adst_p = jnp.pad(a_dst, ((0, 64), (0, 0)))
    out = _sc_msg_fn()(src_p, dst_p, a_src, adst_p, x_src)
    return out[:, :_CH, :].reshape(N, HID)


_BN = 2000
_GRID = N // _BN


def _proj_body(x_ref, w_ref, b_ref, wa_ref, h_ref, a_ref):
    h = jnp.dot(x_ref[...], w_ref[...], preferred_element_type=f32)
    h = h + b_ref[...]
    h_ref[...] = h
    a_ref[...] = jnp.dot(h, wa_ref[...], preferred_element_type=f32)


def _proj(x, w, b, wa):
    ka = wa.shape[1]
    return pl.pallas_call(
        _proj_body,
        grid=(_GRID,),
        in_specs=[
            pl.BlockSpec((_BN, HID), lambda i: (i, 0)),
            pl.BlockSpec((HID, HID), lambda i: (0, 0)),
            pl.BlockSpec((1, HID), lambda i: (0, 0)),
            pl.BlockSpec((HID, ka), lambda i: (0, 0)),
        ],
        out_specs=[
            pl.BlockSpec((_BN, HID), lambda i: (i, 0)),
            pl.BlockSpec((_BN, ka), lambda i: (i, 0)),
        ],
        out_shape=[
            jax.ShapeDtypeStruct((N, HID), f32),
            jax.ShapeDtypeStruct((N, ka), f32),
        ],
    )(x, w, b, wa)


def _tsum_body(a0_ref, a1_ref, kw_ref, kb_ref, o_ref):
    @pl.when(pl.program_id(0) == 0)
    def _():
        o_ref[...] = jnp.zeros_like(o_ref)

    kw = kw_ref[...]
    kb = kb_ref[...]
    t0 = jnp.tanh(jnp.dot(jnp.maximum(a0_ref[...], 0.0), kw,
                          preferred_element_type=f32) + kb)
    t1 = jnp.tanh(jnp.dot(jnp.maximum(a1_ref[...], 0.0), kw,
                          preferred_element_type=f32) + kb)
    o_ref[0:1, :] += jnp.sum(t0, axis=0, keepdims=True)
    o_ref[1:2, :] += jnp.sum(t1, axis=0, keepdims=True)


def _tsum(a0, a1, kw, kb):
    return pl.pallas_call(
        _tsum_body,
        grid=(_GRID,),
        in_specs=[
            pl.BlockSpec((_BN, HID), lambda i: (i, 0)),
            pl.BlockSpec((_BN, HID), lambda i: (i, 0)),
            pl.BlockSpec((HID, HID), lambda i: (0, 0)),
            pl.BlockSpec((1, HID), lambda i: (0, 0)),
        ],
        out_specs=pl.BlockSpec((8, HID), lambda i: (0, 0)),
        out_shape=jax.ShapeDtypeStruct((8, HID), f32),
    )(a0, a1, kw, kb)


def _ln(m, g, b):
    mu = jnp.mean(m, axis=1, keepdims=True)
    d = m - mu
    var = jnp.mean(d * d, axis=1, keepdims=True)
    return d * lax.rsqrt(var + 1e-5) * g + b


def _merge2_body(a0_ref, a1_ref, t_ref, q_ref, g_ref, b_ref, o_ref):
    q = q_ref[...]
    s0 = jnp.sum(q * t_ref[0:1, :]) / N
    s1 = jnp.sum(q * t_ref[1:2, :]) / N
    mx = jnp.maximum(s0, s1)
    e0 = jnp.exp(s0 - mx)
    e1 = jnp.exp(s1 - mx)
    at0 = e0 / (e0 + e1)
    at1 = e1 / (e0 + e1)
    m = at0 * jnp.maximum(a0_ref[...], 0.0) + at1 * jnp.maximum(a1_ref[...], 0.0)
    o_ref[...] = _ln(m, g_ref[...], b_ref[...])


def _merge2(a0, a1, t, q, g, b):
    return pl.pallas_call(
        _merge2_body,
        grid=(_GRID,),
        in_specs=[
            pl.BlockSpec((_BN, HID), lambda i: (i, 0)),
            pl.BlockSpec((_BN, HID), lambda i: (i, 0)),
            pl.BlockSpec((8, HID), lambda i: (0, 0)),
            pl.BlockSpec((1, HID), lambda i: (0, 0)),
            pl.BlockSpec((1, HID), lambda i: (0, 0)),
            pl.BlockSpec((1, HID), lambda i: (0, 0)),
        ],
        out_specs=pl.BlockSpec((_BN, HID), lambda i: (i, 0)),
        out_shape=jax.ShapeDtypeStruct((N, HID), f32),
    )(a0, a1, t, q, g, b)


def _merge1_body(a0_ref, g_ref, b_ref, o_ref):
    o_ref[...] = _ln(jnp.maximum(a0_ref[...], 0.0), g_ref[...], b_ref[...])


def _merge1(a0, g, b):
    return pl.pallas_call(
        _merge1_body,
        grid=(_GRID,),
        in_specs=[
            pl.BlockSpec((_BN, HID), lambda i: (i, 0)),
            pl.BlockSpec((1, HID), lambda i: (0, 0)),
            pl.BlockSpec((1, HID), lambda i: (0, 0)),
        ],
        out_specs=pl.BlockSpec((_BN, HID), lambda i: (i, 0)),
        out_shape=jax.ShapeDtypeStruct((N, HID), f32),
    )(a0, g, b)


def _head_body(r0_ref, r1_ref, fw_ref, fpw_ref, fpb_ref, w1_ref, b1_ref,
               w2_ref, b2_ref, o_ref):
    wsum = fw_ref[0, 0] + fw_ref[0, 1]
    z = (fw_ref[0, 0] * r0_ref[...] + fw_ref[0, 1] * r1_ref[...]) / wsum
    z = jnp.maximum(jnp.dot(z, fpw_ref[...], preferred_element_type=f32)
                    + fpb_ref[...], 0.0)
    z = jnp.maximum(jnp.dot(z, w1_ref[...], preferred_element_type=f32)
                    + b1_ref[...], 0.0)
    o_ref[...] = jnp.dot(z, w2_ref[...], preferred_element_type=f32) + b2_ref[...]


def _head(r0, r1, fw, fpw, fpb, w1, b1, w2, b2):
    return pl.pallas_call(
        _head_body,
        grid=(_GRID,),
        in_specs=[
            pl.BlockSpec((_BN, HID), lambda i: (i, 0)),
            pl.BlockSpec((_BN, HID), lambda i: (i, 0)),
            pl.BlockSpec((1, HID), lambda i: (0, 0)),
            pl.BlockSpec((HID, HID), lambda i: (0, 0)),
            pl.BlockSpec((1, HID), lambda i: (0, 0)),
            pl.BlockSpec((HID, HID), lambda i: (0, 0)),
            pl.BlockSpec((1, HID), lambda i: (0, 0)),
            pl.BlockSpec((HID, OUT), lambda i: (0, 0)),
            pl.BlockSpec((1, OUT), lambda i: (0, 0)),
        ],
        out_specs=pl.BlockSpec((_BN, OUT), lambda i: (i, 0)),
        out_shape=jax.ShapeDtypeStruct((N, OUT), f32),
    )(r0, r1, fw, fpw, fpb, w1, b1, w2, b2)


def _band(lin):
    eye = jnp.eye(HEADS, 16, dtype=f32)
    w3 = jnp.einsum('hd,hk->hdk', lin, eye)
    return w3.reshape(HID, 16)


def _row128(*vals):
    v = jnp.stack([v.astype(f32) for v in vals])
    return jnp.pad(v, (0, 128 - v.shape[0])).reshape(1, 128)


def kernel(x_snp, x_gene, edge_sg, edge_gs, edge_gg, params):
    p = params
    edges = {'sg': edge_sg, 'gs': edge_gs, 'gg': edge_gg}
    xd = {'snp': x_snp, 'gene': x_gene}
    louts = []
    for i in range(2):
        wa_snp = jnp.concatenate(
            [_band(p['lin_src_sg_%d' % i]), _band(p['lin_dst_gs_%d' % i])],
            axis=1)
        wa_gene = jnp.concatenate(
            [_band(p['lin_dst_sg_%d' % i]), _band(p['lin_src_gs_%d' % i]),
             _band(p['lin_src_gg_%d' % i]), _band(p['lin_dst_gg_%d' % i])],
            axis=1)
        h_snp, a_snp = _proj(xd['snp'], p['proj_w_snp_%d' % i],
                             p['proj_b_snp_%d' % i].reshape(1, HID), wa_snp)
        h_gene, a_gene = _proj(xd['gene'], p['proj_w_gene_%d' % i],
                               p['proj_b_gene_%d' % i].reshape(1, HID), wa_gene)
        a_src_sg, a_dst_gs = a_snp[:, 0:16], a_snp[:, 16:32]
        a_dst_sg, a_src_gs = a_gene[:, 0:16], a_gene[:, 16:32]
        a_src_gg, a_dst_gg = a_gene[:, 32:48], a_gene[:, 48:64]

        agg_sg = _sc_messages(edges['sg'][0], edges['sg'][1],
                              a_src_sg, a_dst_sg, h_snp)
        agg_gs = _sc_messages(edges['gs'][0], edges['gs'][1],
                              a_src_gs, a_dst_gs, h_gene)
        agg_gg = _sc_messages(edges['gg'][0], edges['gg'][1],
                              a_src_gg, a_dst_gg, h_gene)

        kw = p['k_w_%d' % i]
        kb = p['k_b_%d' % i].reshape(1, HID)
        g = p['ln_g_%d' % i].reshape(1, HID)
        b = p['ln_b_%d' % i].reshape(1, HID)
        t = _tsum(agg_sg, agg_gg, kw, kb)
        res_gene = _merge2(agg_sg, agg_gg, t, p['q_%d' % i].reshape(1, HID),
                           g, b)
        res_snp = _merge1(agg_gs, g, b)
        xd = {'snp': res_snp, 'gene': res_gene}
        louts.append(xd)

    fw = _row128(p['fusion_w'][0], p['fusion_w'][1])
    args = (fw, p['fp_w'], p['fp_b'].reshape(1, HID),
            p['ow1'], p['ob1'].reshape(1, HID),
            p['ow2'], p['ob2'].reshape(1, OUT))
    o_snp = _head(louts[0]['snp'], louts[1]['snp'], *args)
    o_gene = _head(louts[0]['gene'], louts[1]['gene'], *args)
    return jnp.concatenate([o_snp, o_gene], axis=0)


# speedup vs baseline: 21.9341x; 1.1285x over previous
import functools

import jax
import jax.numpy as jnp
from jax import lax
from jax.experimental import pallas as pl
from jax.experimental.pallas import tpu as pltpu
from jax.experimental.pallas import tpu_sc as plsc

N = 50000
E = 200000
HID = 128
HEADS = 8
DHEAD = 16
OUT = 64
NSUB = 16
LANES = 16

f32 = jnp.float32
i32 = jnp.int32


def _build_sc_msg(n_nodes, n_edges, nchunk, ch, chp, eb, mb, interpret=False,
                  stage=99):
    kpc = nchunk // 2
    pt = n_edges // NSUB
    cnt = -((-(pt + 4)) // eb) * eb
    cap = pt + 2 * mb
    stripe = chp // NSUB
    zb = 80 if stripe % 80 == 0 else (40 if stripe % 40 == 0 else stripe)
    assert chp % NSUB == 0 and stripe % zb == 0 and eb % 16 == 0 and mb % 16 == 0
    nzb = stripe // zb
    dump = ch

    def body(src_hbm, dst_hbm, asrc_hbm, adst_hbm, x_hbm, out_hbm,
             den_sh, agg_sh,
             csrc, cgd, srcb, dstb,
             asr0, asr1, adr0, adr1, exr, denr, wbuf, xr0, xr1, sidx,
             zbufa, zbufb, sa0, sa1, sb0, sb1, sx0, sx1):
        c = lax.axis_index("c")
        s = lax.axis_index("s")
        z16 = jnp.zeros((16,), f32)
        asrs, adrs, xrs = (asr0, asr1), (adr0, adr1), (xr0, xr1)
        sas, sbs, sxs = (sa0, sa1), (sb0, sb1), (sx0, sx1)

        def zfa(i, _):
            for t in range(8):
                zbufa[i, pl.ds(16 * t, 16)] = z16
            return 0
        lax.fori_loop(0, zb, zfa, 0)

        def zfb(i, _):
            zbufb[i, :] = z16
            return 0
        lax.fori_loop(0, stripe, zfb, 0)

        base = s * stripe
        lane = lax.iota(i32, 16)

        e0 = s * pt
        start = pl.multiple_of(e0 - (e0 % 8), 8)

        def gat(vec, idxv):
            return vec.at[idxv].get(mode="promise_in_bounds")

        for k in range(kpc):
            chunk = kpc * c + k
            lo = chunk * ch
            hi = lo + ch

            for t in range(nzb):
                pltpu.sync_copy(zbufa, agg_sh.at[pl.ds(base + zb * t, zb)])
            pltpu.sync_copy(zbufb, den_sh.at[pl.ds(base, stripe)])
            plsc.subcore_barrier()

            def scan_batch(bb, pos0):
              ebase = pl.multiple_of(start + bb * eb, 8)
              pltpu.sync_copy(src_hbm.at[pl.ds(ebase, eb)], srcb)
              pltpu.sync_copy(dst_hbm.at[pl.ds(ebase, eb)], dstb)

              def grp(g, pos):
                sv = srcb[pl.ds(g * 16, 16)]
                dv = dstb[pl.ds(g * 16, 16)]
                eid = ebase + g * 16 + lane
                m = ((eid >= e0) & (eid < e0 + pt)
                     & (dv >= lo) & (dv < hi))
                mi = jnp.where(m, 1, 0)
                x = mi
                for sb in (1, 2, 4, 8):
                    sh = gat(x, jnp.maximum(lane - sb, 0))
                    x = x + jnp.where(lane >= sb, sh, 0)
                tot = x[15]
                r = jnp.where(m, lane - (x - mi), 0)
                vi = mi
                cx, dx = sv, dv
                for sb in (1, 2, 4, 8):
                    si = jnp.minimum(lane + sb, 15)
                    cxs = gat(cx, si)
                    dxs = gat(dx, si)
                    rs = gat(r, si)
                    vs = gat(vi, si)
                    take = (lane + sb <= 15) & (vs > 0) & ((rs & sb) != 0)
                    keep = (vi > 0) & ((r & sb) == 0)
                    cx = jnp.where(take, cxs, cx)
                    dx = jnp.where(take, dxs, dx)
                    r = jnp.where(take, rs - sb, r)
                    vi = jnp.where(take, 1, jnp.where(keep, 1, 0))
                csrc[pl.ds(pos, 16)] = cx
                cgd[pl.ds(pos, 16)] = dx
                return pos + tot

              return lax.fori_loop(0, eb // 16, grp, pos0)

            pos = lax.fori_loop(0, cnt // eb, scan_batch, 0)

            gpad = jnp.full((16,), lo + dump, i32)
            zpad = jnp.zeros((16,), i32)
            for t in range(mb // 16):
                csrc[pl.ds(pos + 16 * t, 16)] = zpad
                cgd[pl.ds(pos + 16 * t, 16)] = gpad
            nbat = (pos + mb - 1) // mb

            def fill_sidx(off):
                def cp(j, _):
                    sidx[pl.ds(j * 16, 16)] = cgd[pl.ds(off + j * 16, 16)] - lo
                    return 0
                lax.fori_loop(0, mb // 16, cp, 0)

            def a_start(par, b):
                @pl.when(b < nbat)
                def _():
                    off = b * mb
                    pltpu.async_copy(
                        asrc_hbm.at[csrc.at[pl.ds(off, mb)]], asrs[par],
                        sas[par])
                    pltpu.async_copy(
                        adst_hbm.at[cgd.at[pl.ds(off, mb)]], adrs[par],
                        sbs[par])

            def a_wait(par, b):
                off = b * mb
                pltpu.make_async_copy(
                    asrc_hbm.at[csrc.at[pl.ds(off, mb)]], asrs[par],
                    sas[par]).wait()
                pltpu.make_async_copy(
                    adst_hbm.at[cgd.at[pl.ds(off, mb)]], adrs[par],
                    sbs[par]).wait()

            a_start(0, 0)

            def den_pair(t, _):
                for par in (0, 1):
                    b = 2 * t + par
                    a_start(1 - par, b + 1)

                    @pl.when(b < nbat)
                    def _():
                        off = b * mb
                        a_wait(par, b)
                        asr, adr = asrs[par], adrs[par]

                        def exrow(j, _):
                            av = asr[j, :] + adr[j, :]
                            av = jnp.where(av > 0, av, 0.2 * av)
                            exr[j, :] = jnp.exp(av)
                            return 0
                        lax.fori_loop(0, mb, exrow, 0)
                        fill_sidx(off)
                        pltpu.sync_copy(exr, den_sh.at[sidx], add=True)
                return 0
            lax.fori_loop(0, (nbat + 1) // 2, den_pair, 0)
            plsc.subcore_barrier()

            def m_start(par, b):
                @pl.when(b < nbat)
                def _():
                    off = b * mb
                    pltpu.async_copy(
                        x_hbm.at[csrc.at[pl.ds(off, mb)]], xrs[par], sxs[par])
                a_start(par, b)

            m_start(0, 0)

            def msg_pair(t, _):
                for par in (0, 1):
                    b = 2 * t + par
                    m_start(1 - par, b + 1)

                    @pl.when(b < nbat)
                    def _():
                        off = b * mb
                        fill_sidx(off)
                        pltpu.sync_copy(den_sh.at[sidx], denr)
                        a_wait(par, b)
                        asr, adr, xr = asrs[par], adrs[par], xrs[par]

                        def wrow(j, _):
                            av = asr[j, :] + adr[j, :]
                            av = jnp.where(av > 0, av, 0.2 * av)
                            wbuf[j, :] = jnp.exp(av) / (denr[j, :] + 1e-16)
                            return 0
                        lax.fori_loop(0, mb, wrow, 0)
                        pltpu.make_async_copy(
                            x_hbm.at[csrc.at[pl.ds(off, mb)]], xr,
                            sxs[par]).wait()

                        def scale(j, _):
                            wv = wbuf[j, :]
                            for h in range(HEADS):
                                xr[j, pl.ds(h * 16, 16)] = (
                                    xr[j, pl.ds(h * 16, 16)] * wv[h])
                            return 0
                        lax.fori_loop(0, mb, scale, 0)
                        pltpu.sync_copy(xr, agg_sh.at[sidx], add=True)
                return 0
            lax.fori_loop(0, (nbat + 1) // 2, msg_pair, 0)
            plsc.subcore_barrier()

            for t in range(nzb):
                pltpu.sync_copy(agg_sh.at[pl.ds(base + zb * t, zb)],
                                out_hbm.at[chunk, pl.ds(base + zb * t, zb)])
            plsc.subcore_barrier()

    mesh = plsc.VectorSubcoreMesh(core_axis_name="c", subcore_axis_name="s",
                                  num_cores=2, num_subcores=NSUB)
    fn = pl.kernel(
        body,
        out_type=jax.ShapeDtypeStruct((nchunk, chp, HID), f32),
        mesh=mesh,
        scratch_types=dict(
            den_sh=pltpu.VMEM_SHARED((chp, 16), f32),
            agg_sh=pltpu.VMEM_SHARED((chp, HID), f32),
            csrc=pltpu.VMEM((cap,), i32),
            cgd=pltpu.VMEM((cap,), i32),
            srcb=pltpu.VMEM((eb,), i32),
            dstb=pltpu.VMEM((eb,), i32),
            asr0=pltpu.VMEM((mb, 16), f32),
            asr1=pltpu.VMEM((mb, 16), f32),
            adr0=pltpu.VMEM((mb, 16), f32),
            adr1=pltpu.VMEM((mb, 16), f32),
            exr=pltpu.VMEM((mb, 16), f32),
            denr=pltpu.VMEM((mb, 16), f32),
            wbuf=pltpu.VMEM((mb, 16), f32),
            xr0=pltpu.VMEM((mb, HID), f32),
            xr1=pltpu.VMEM((mb, HID), f32),
            sidx=pltpu.VMEM((mb,), i32),
            zbufa=pltpu.VMEM((zb, HID), f32),
            zbufb=pltpu.VMEM((stripe, 16), f32),
            sa0=pltpu.SemaphoreType.DMA,
            sa1=pltpu.SemaphoreType.DMA,
            sb0=pltpu.SemaphoreType.DMA,
            sb1=pltpu.SemaphoreType.DMA,
            sx0=pltpu.SemaphoreType.DMA,
            sx1=pltpu.SemaphoreType.DMA,
        ),
        compiler_params=pltpu.CompilerParams(use_tc_tiling_on_sc=False),
        interpret=interpret,
    )
    return fn


_NCHUNK = 16
_CH = N // _NCHUNK
_CHP = 3200


@functools.cache
def _sc_msg_fn():
    return _build_sc_msg(N, E, _NCHUNK, _CH, _CHP, 1600, 128)


_EDGE_PAD = (15 * (E // NSUB) - 4 + 12800) - E


def _sc_messages(src, dst, a_src, a_dst, x_src):
    src_p = jnp.pad(src, (0, _EDGE_PAD))
    dst_p = jnp.pad(dst, (0, _EDGE_PAD))
    adst_p = jnp.pad(a_dst, ((0, 64), (0, 0)))
    out = _sc_msg_fn()(src_p, dst_p, a_src, adst_p, x_src)
    return out[:, :_CH, :].reshape(N, HID)


_BN = 2000
_GRID = N // _BN


def _proj_body(x_ref, w_ref, b_ref, wa_ref, h_ref, a_ref):
    h = jnp.dot(x_ref[...], w_ref[...], preferred_element_type=f32)
    h = h + b_ref[...]
    h_ref[...] = h
    a_ref[...] = jnp.dot(h, wa_ref[...], preferred_element_type=f32)


def _proj(x, w, b, wa):
    ka = wa.shape[1]
    return pl.pallas_call(
        _proj_body,
        grid=(_GRID,),
        in_specs=[
            pl.BlockSpec((_BN, HID), lambda i: (i, 0)),
            pl.BlockSpec((HID, HID), lambda i: (0, 0)),
            pl.BlockSpec((1, HID), lambda i: (0, 0)),
            pl.BlockSpec((HID, ka), lambda i: (0, 0)),
        ],
        out_specs=[
            pl.BlockSpec((_BN, HID), lambda i: (i, 0)),
            pl.BlockSpec((_BN, ka), lambda i: (i, 0)),
        ],
        out_shape=[
            jax.ShapeDtypeStruct((N, HID), f32),
            jax.ShapeDtypeStruct((N, ka), f32),
        ],
    )(x, w, b, wa)


def _tsum_body(a0_ref, a1_ref, kw_ref, kb_ref, o_ref):
    @pl.when(pl.program_id(0) == 0)
    def _():
        o_ref[...] = jnp.zeros_like(o_ref)

    kw = kw_ref[...]
    kb = kb_ref[...]
    t0 = jnp.tanh(jnp.dot(jnp.maximum(a0_ref[...], 0.0), kw,
                          preferred_element_type=f32) + kb)
    t1 = jnp.tanh(jnp.dot(jnp.maximum(a1_ref[...], 0.0), kw,
                          preferred_element_type=f32) + kb)
    o_ref[0:1, :] += jnp.sum(t0, axis=0, keepdims=True)
    o_ref[1:2, :] += jnp.sum(t1, axis=0, keepdims=True)


def _tsum(a0, a1, kw, kb):
    return pl.pallas_call(
        _tsum_body,
        grid=(_GRID,),
        in_specs=[
            pl.BlockSpec((_BN, HID), lambda i: (i, 0)),
            pl.BlockSpec((_BN, HID), lambda i: (i, 0)),
            pl.BlockSpec((HID, HID), lambda i: (0, 0)),
            pl.BlockSpec((1, HID), lambda i: (0, 0)),
        ],
        out_specs=pl.BlockSpec((8, HID), lambda i: (0, 0)),
        out_shape=jax.ShapeDtypeStruct((8, HID), f32),
    )(a0, a1, kw, kb)


def _ln(m, g, b):
    mu = jnp.mean(m, axis=1, keepdims=True)
    d = m - mu
    var = jnp.mean(d * d, axis=1, keepdims=True)
    return d * lax.rsqrt(var + 1e-5) * g + b


def _merge2_body(a0_ref, a1_ref, t_ref, q_ref, g_ref, b_ref, o_ref):
    q = q_ref[...]
    s0 = jnp.sum(q * t_ref[0:1, :]) / N
    s1 = jnp.sum(q * t_ref[1:2, :]) / N
    mx = jnp.maximum(s0, s1)
    e0 = jnp.exp(s0 - mx)
    e1 = jnp.exp(s1 - mx)
    at0 = e0 / (e0 + e1)
    at1 = e1 / (e0 + e1)
    m = at0 * jnp.maximum(a0_ref[...], 0.0) + at1 * jnp.maximum(a1_ref[...], 0.0)
    o_ref[...] = _ln(m, g_ref[...], b_ref[...])


def _merge2(a0, a1, t, q, g, b):
    return pl.pallas_call(
        _merge2_body,
        grid=(_GRID,),
        in_specs=[
            pl.BlockSpec((_BN, HID), lambda i: (i, 0)),
            pl.BlockSpec((_BN, HID), lambda i: (i, 0)),
            pl.BlockSpec((8, HID), lambda i: (0, 0)),
            pl.BlockSpec((1, HID), lambda i: (0, 0)),
            pl.BlockSpec((1, HID), lambda i: (0, 0)),
            pl.BlockSpec((1, HID), lambda i: (0, 0)),
        ],
        out_specs=pl.BlockSpec((_BN, HID), lambda i: (i, 0)),
        out_shape=jax.ShapeDtypeStruct((N, HID), f32),
    )(a0, a1, t, q, g, b)


def _merge1_body(a0_ref, g_ref, b_ref, o_ref):
    o_ref[...] = _ln(jnp.maximum(a0_ref[...], 0.0), g_ref[...], b_ref[...])


def _merge1(a0, g, b):
    return pl.pallas_call(
        _merge1_body,
        grid=(_GRID,),
        in_specs=[
            pl.BlockSpec((_BN, HID), lambda i: (i, 0)),
            pl.BlockSpec((1, HID), lambda i: (0, 0)),
            pl.BlockSpec((1, HID), lambda i: (0, 0)),
        ],
        out_specs=pl.BlockSpec((_BN, HID), lambda i: (i, 0)),
        out_shape=jax.ShapeDtypeStruct((N, HID), f32),
    )(a0, g, b)


def _head_body(r0_ref, r1_ref, fw_ref, fpw_ref, fpb_ref, w1_ref, b1_ref,
               w2_ref, b2_ref, o_ref):
    wsum = fw_ref[0, 0] + fw_ref[0, 1]
    z = (fw_ref[0, 0] * r0_ref[...] + fw_ref[0, 1] * r1_ref[...]) / wsum
    z = jnp.maximum(jnp.dot(z, fpw_ref[...], preferred_element_type=f32)
                    + fpb_ref[...], 0.0)
    z = jnp.maximum(jnp.dot(z, w1_ref[...], preferred_element_type=f32)
                    + b1_ref[...], 0.0)
    o_ref[...] = jnp.dot(z, w2_ref[...], preferred_element_type=f32) + b2_ref[...]


def _head(r0, r1, fw, fpw, fpb, w1, b1, w2, b2):
    return pl.pallas_call(
        _head_body,
        grid=(_GRID,),
        in_specs=[
            pl.BlockSpec((_BN, HID), lambda i: (i, 0)),
            pl.BlockSpec((_BN, HID), lambda i: (i, 0)),
            pl.BlockSpec((1, HID), lambda i: (0, 0)),
            pl.BlockSpec((HID, HID), lambda i: (0, 0)),
            pl.BlockSpec((1, HID), lambda i: (0, 0)),
            pl.BlockSpec((HID, HID), lambda i: (0, 0)),
            pl.BlockSpec((1, HID), lambda i: (0, 0)),
            pl.BlockSpec((HID, OUT), lambda i: (0, 0)),
            pl.BlockSpec((1, OUT), lambda i: (0, 0)),
        ],
        out_specs=pl.BlockSpec((_BN, OUT), lambda i: (i, 0)),
        out_shape=jax.ShapeDtypeStruct((N, OUT), f32),
    )(r0, r1, fw, fpw, fpb, w1, b1, w2, b2)


def _band(lin):
    eye = jnp.eye(HEADS, 16, dtype=f32)
    w3 = jnp.einsum('hd,hk->hdk', lin, eye)
    return w3.reshape(HID, 16)


def _row128(*vals):
    v = jnp.stack([v.astype(f32) for v in vals])
    return jnp.pad(v, (0, 128 - v.shape[0])).reshape(1, 128)


def kernel(x_snp, x_gene, edge_sg, edge_gs, edge_gg, params):
    p = params
    edges = {'sg': edge_sg, 'gs': edge_gs, 'gg': edge_gg}
    xd = {'snp': x_snp, 'gene': x_gene}
    louts = []
    for i in range(2):
        wa_snp = jnp.concatenate(
            [_band(p['lin_src_sg_%d' % i]), _band(p['lin_dst_gs_%d' % i])],
            axis=1)
        wa_gene = jnp.concatenate(
            [_band(p['lin_dst_sg_%d' % i]), _band(p['lin_src_gs_%d' % i]),
             _band(p['lin_src_gg_%d' % i]), _band(p['lin_dst_gg_%d' % i])],
            axis=1)
        h_snp, a_snp = _proj(xd['snp'], p['proj_w_snp_%d' % i],
                             p['proj_b_snp_%d' % i].reshape(1, HID), wa_snp)
        h_gene, a_gene = _proj(xd['gene'], p['proj_w_gene_%d' % i],
                               p['proj_b_gene_%d' % i].reshape(1, HID), wa_gene)
        a_src_sg, a_dst_gs = a_snp[:, 0:16], a_snp[:, 16:32]
        a_dst_sg, a_src_gs = a_gene[:, 0:16], a_gene[:, 16:32]
        a_src_gg, a_dst_gg = a_gene[:, 32:48], a_gene[:, 48:64]

        agg_sg = _sc_messages(edges['sg'][0], edges['sg'][1],
                              a_src_sg, a_dst_sg, h_snp)
        agg_gs = _sc_messages(edges['gs'][0], edges['gs'][1],
                              a_src_gs, a_dst_gs, h_gene)
        agg_gg = _sc_messages(edges['gg'][0], edges['gg'][1],
                              a_src_gg, a_dst_gg, h_gene)

        kw = p['k_w_%d' % i]
        kb = p['k_b_%d' % i].reshape(1, HID)
        g = p['ln_g_%d' % i].reshape(1, HID)
        b = p['ln_b_%d' % i].reshape(1, HID)
        t = _tsum(agg_sg, agg_gg, kw, kb)
        res_gene = _merge2(agg_sg, agg_gg, t, p['q_%d' % i].reshape(1, HID),
                           g, b)
        res_snp = _merge1(agg_gs, g, b)
        xd = {'snp': res_snp, 'gene': res_gene}
        louts.append(xd)

    fw = _row128(p['fusion_w'][0], p['fusion_w'][1])
    args = (fw, p['fp_w'], p['fp_b'].reshape(1, HID),
            p['ow1'], p['ob1'].reshape(1, HID),
            p['ow2'], p['ob2'].reshape(1, OUT))
    o_snp = _head(louts[0]['snp'], louts[1]['snp'], *args)
    o_gene = _head(louts[0]['gene'], louts[1]['gene'], *args)
    return jnp.concatenate([o_snp, o_gene], axis=0)


# speedup vs baseline: 26.4703x; 1.2068x over previous
import functools

import jax
import jax.numpy as jnp
from jax import lax
from jax.experimental import pallas as pl
from jax.experimental.pallas import tpu as pltpu
from jax.experimental.pallas import tpu_sc as plsc

N = 50000
E = 200000
HID = 128
HEADS = 8
DHEAD = 16
OUT = 64
NSUB = 16
LANES = 16

f32 = jnp.float32
i32 = jnp.int32


def _build_sc_msg(n_nodes, n_edges, nchunk, ch, chp, eb, mb, interpret=False,
                  stage=99):
    kpc = nchunk // 2
    pt = n_edges // NSUB
    cnt = -((-(pt + 4)) // eb) * eb
    cap = pt + 2 * mb
    stripe = chp // NSUB
    zb = 80 if stripe % 80 == 0 else (40 if stripe % 40 == 0 else stripe)
    assert chp % NSUB == 0 and stripe % zb == 0 and eb % 16 == 0 and mb % 16 == 0
    nzb = stripe // zb
    dump = ch

    def body(src_hbm, dst_hbm, asrc_hbm, adst_hbm, x_hbm, out_hbm,
             den_sh, agg_sh,
             csrc, cgd, srcb, dstb,
             asr0, asr1, adr0, adr1, exr, denr, wbuf, xr0, xr1, xout, sidx,
             zbufa, zbufb, sa0, sa1, sb0, sb1, sx0, sx1):
        c = lax.axis_index("c")
        s = lax.axis_index("s")
        z16 = jnp.zeros((16,), f32)
        HIMASK = jnp.int32(-65536)
        asrs, adrs, xrs = (asr0, asr1), (adr0, adr1), (xr0, xr1)
        sas, sbs, sxs = (sa0, sa1), (sb0, sb1), (sx0, sx1)

        def zfa(i, _):
            for t in range(8):
                zbufa[i, pl.ds(16 * t, 16)] = z16
            return 0
        lax.fori_loop(0, zb, zfa, 0)

        def zfb(i, _):
            zbufb[i, :] = z16
            return 0
        lax.fori_loop(0, stripe, zfb, 0)

        base = s * stripe
        lane = lax.iota(i32, 16)

        e0 = s * pt
        start = pl.multiple_of(e0 - (e0 % 8), 8)

        def gat(vec, idxv):
            return vec.at[idxv].get(mode="promise_in_bounds")

        for k in range(kpc):
            chunk = kpc * c + k
            lo = chunk * ch
            hi = lo + ch

            for t in range(nzb):
                pltpu.sync_copy(zbufa, agg_sh.at[pl.ds(base + zb * t, zb)])
            pltpu.sync_copy(zbufb, den_sh.at[pl.ds(base, stripe)])
            plsc.subcore_barrier()

            def scan_batch(bb, pos0):
              ebase = pl.multiple_of(start + bb * eb, 8)
              pltpu.sync_copy(src_hbm.at[pl.ds(ebase, eb)], srcb)
              pltpu.sync_copy(dst_hbm.at[pl.ds(ebase, eb)], dstb)

              def grp(g, pos):
                sv = srcb[pl.ds(g * 16, 16)]
                dv = dstb[pl.ds(g * 16, 16)]
                eid = ebase + g * 16 + lane
                m = ((eid >= e0) & (eid < e0 + pt)
                     & (dv >= lo) & (dv < hi))
                mi = jnp.where(m, 1, 0)
                x = mi
                for sb in (1, 2, 4, 8):
                    sh = gat(x, jnp.maximum(lane - sb, 0))
                    x = x + jnp.where(lane >= sb, sh, 0)
                tot = x[15]
                r = jnp.where(m, lane - (x - mi), 0)
                vi = mi
                cx, dx = sv, dv
                for sb in (1, 2, 4, 8):
                    si = jnp.minimum(lane + sb, 15)
                    cxs = gat(cx, si)
                    dxs = gat(dx, si)
                    rs = gat(r, si)
                    vs = gat(vi, si)
                    take = (lane + sb <= 15) & (vs > 0) & ((rs & sb) != 0)
                    keep = (vi > 0) & ((r & sb) == 0)
                    cx = jnp.where(take, cxs, cx)
                    dx = jnp.where(take, dxs, dx)
                    r = jnp.where(take, rs - sb, r)
                    vi = jnp.where(take, 1, jnp.where(keep, 1, 0))
                csrc[pl.ds(pos, 16)] = cx
                cgd[pl.ds(pos, 16)] = dx
                return pos + tot

              return lax.fori_loop(0, eb // 16, grp, pos0)

            pos = lax.fori_loop(0, cnt // eb, scan_batch, 0)

            gpad = jnp.full((16,), lo + dump, i32)
            zpad = jnp.zeros((16,), i32)
            for t in range(mb // 16):
                csrc[pl.ds(pos + 16 * t, 16)] = zpad
                cgd[pl.ds(pos + 16 * t, 16)] = gpad
            nbat = (pos + mb - 1) // mb

            def fill_sidx(off):
                def cp(j, _):
                    sidx[pl.ds(j * 16, 16)] = cgd[pl.ds(off + j * 16, 16)] - lo
                    return 0
                lax.fori_loop(0, mb // 16, cp, 0)

            def a_start(par, b):
                @pl.when(b < nbat)
                def _():
                    off = b * mb
                    pltpu.async_copy(
                        asrc_hbm.at[csrc.at[pl.ds(off, mb)]], asrs[par],
                        sas[par])
                    pltpu.async_copy(
                        adst_hbm.at[cgd.at[pl.ds(off, mb)]], adrs[par],
                        sbs[par])

            def a_wait(par, b):
                off = b * mb
                pltpu.make_async_copy(
                    asrc_hbm.at[csrc.at[pl.ds(off, mb)]], asrs[par],
                    sas[par]).wait()
                pltpu.make_async_copy(
                    adst_hbm.at[cgd.at[pl.ds(off, mb)]], adrs[par],
                    sbs[par]).wait()

            a_start(0, 0)

            def den_pair(t, _):
                for par in (0, 1):
                    b = 2 * t + par
                    a_start(1 - par, b + 1)

                    @pl.when(b < nbat)
                    def _():
                        off = b * mb
                        a_wait(par, b)
                        asr, adr = asrs[par], adrs[par]

                        def exrow(j, _):
                            av = asr[j, :] + adr[j, :]
                            av = jnp.where(av > 0, av, 0.2 * av)
                            exr[j, :] = jnp.exp(av)
                            return 0
                        lax.fori_loop(0, mb, exrow, 0)
                        fill_sidx(off)
                        pltpu.sync_copy(exr, den_sh.at[sidx], add=True)
                return 0
            lax.fori_loop(0, (nbat + 1) // 2, den_pair, 0)
            plsc.subcore_barrier()

            def m_start(par, b):
                @pl.when(b < nbat)
                def _():
                    off = b * mb
                    pltpu.async_copy(
                        x_hbm.at[csrc.at[pl.ds(off, mb)]], xrs[par], sxs[par])
                a_start(par, b)

            m_start(0, 0)

            def msg_pair(t, _):
                for par in (0, 1):
                    b = 2 * t + par
                    m_start(1 - par, b + 1)

                    @pl.when(b < nbat)
                    def _():
                        off = b * mb
                        fill_sidx(off)
                        pltpu.sync_copy(den_sh.at[sidx], denr)
                        a_wait(par, b)
                        asr, adr, xr = asrs[par], adrs[par], xrs[par]

                        def wrow(j, _):
                            av = asr[j, :] + adr[j, :]
                            av = jnp.where(av > 0, av, 0.2 * av)
                            wbuf[j, :] = jnp.exp(av) / (denr[j, :] + 1e-16)
                            return 0
                        lax.fori_loop(0, mb, wrow, 0)
                        pltpu.make_async_copy(
                            x_hbm.at[csrc.at[pl.ds(off, mb)]], xr,
                            sxs[par]).wait()

                        def scale(j, _):
                            wv = wbuf[j, :]
                            for t in range(4):
                                w16 = xr[j, pl.ds(16 * t, 16)]
                                lof = lax.bitcast_convert_type(
                                    lax.shift_left(w16, 16), f32)
                                hif = lax.bitcast_convert_type(
                                    w16 & HIMASK, f32)
                                xout[j, pl.ds(16 * t, 16)] = lof * wv[t]
                                xout[j, pl.ds(64 + 16 * t, 16)] = (
                                    hif * wv[t + 4])
                            return 0
                        lax.fori_loop(0, mb, scale, 0)
                        pltpu.sync_copy(xout, agg_sh.at[sidx], add=True)
                return 0
            lax.fori_loop(0, (nbat + 1) // 2, msg_pair, 0)
            plsc.subcore_barrier()

            for t in range(nzb):
                pltpu.sync_copy(agg_sh.at[pl.ds(base + zb * t, zb)],
                                out_hbm.at[chunk, pl.ds(base + zb * t, zb)])
            plsc.subcore_barrier()

    mesh = plsc.VectorSubcoreMesh(core_axis_name="c", subcore_axis_name="s",
                                  num_cores=2, num_subcores=NSUB)
    fn = pl.kernel(
        body,
        out_type=jax.ShapeDtypeStruct((nchunk, chp, HID), f32),
        mesh=mesh,
        scratch_types=dict(
            den_sh=pltpu.VMEM_SHARED((chp, 16), f32),
            agg_sh=pltpu.VMEM_SHARED((chp, HID), f32),
            csrc=pltpu.VMEM((cap,), i32),
            cgd=pltpu.VMEM((cap,), i32),
            srcb=pltpu.VMEM((eb,), i32),
            dstb=pltpu.VMEM((eb,), i32),
            asr0=pltpu.VMEM((mb, 16), f32),
            asr1=pltpu.VMEM((mb, 16), f32),
            adr0=pltpu.VMEM((mb, 16), f32),
            adr1=pltpu.VMEM((mb, 16), f32),
            exr=pltpu.VMEM((mb, 16), f32),
            denr=pltpu.VMEM((mb, 16), f32),
            wbuf=pltpu.VMEM((mb, 16), f32),
            xr0=pltpu.VMEM((mb, HID // 2), i32),
            xr1=pltpu.VMEM((mb, HID // 2), i32),
            xout=pltpu.VMEM((mb, HID), f32),
            sidx=pltpu.VMEM((mb,), i32),
            zbufa=pltpu.VMEM((zb, HID), f32),
            zbufb=pltpu.VMEM((stripe, 16), f32),
            sa0=pltpu.SemaphoreType.DMA,
            sa1=pltpu.SemaphoreType.DMA,
            sb0=pltpu.SemaphoreType.DMA,
            sb1=pltpu.SemaphoreType.DMA,
            sx0=pltpu.SemaphoreType.DMA,
            sx1=pltpu.SemaphoreType.DMA,
        ),
        compiler_params=pltpu.CompilerParams(use_tc_tiling_on_sc=False),
        interpret=interpret,
    )
    return fn


_NCHUNK = 16
_CH = N // _NCHUNK
_CHP = 3200


@functools.cache
def _sc_msg_fn():
    return _build_sc_msg(N, E, _NCHUNK, _CH, _CHP, 1600, 128)


_EDGE_PAD = (15 * (E // NSUB) - 4 + 12800) - E


def _sc_messages(src, dst, a_src, a_dst, x_src):
    src_p = jnp.pad(src, (0, _EDGE_PAD))
    dst_p = jnp.pad(dst, (0, _EDGE_PAD))
    adst_p = jnp.pad(a_dst, ((0, 64), (0, 0)))
    hb = x_src.astype(jnp.bfloat16)
    hp = jnp.stack([hb[:, :64], hb[:, 64:]], axis=-1)
    xi = jax.lax.bitcast_convert_type(hp, i32)
    out = _sc_msg_fn()(src_p, dst_p, a_src, adst_p, xi)
    return out[:, :_CH, :].reshape(N, HID)


_BN = 2000
_GRID = N // _BN


def _proj_body(x_ref, w_ref, b_ref, wa_ref, h_ref, a_ref):
    h = jnp.dot(x_ref[...], w_ref[...], preferred_element_type=f32)
    h = h + b_ref[...]
    h_ref[...] = h
    a_ref[...] = jnp.dot(h, wa_ref[...], preferred_element_type=f32)


def _proj(x, w, b, wa):
    ka = wa.shape[1]
    return pl.pallas_call(
        _proj_body,
        grid=(_GRID,),
        in_specs=[
            pl.BlockSpec((_BN, HID), lambda i: (i, 0)),
            pl.BlockSpec((HID, HID), lambda i: (0, 0)),
            pl.BlockSpec((1, HID), lambda i: (0, 0)),
            pl.BlockSpec((HID, ka), lambda i: (0, 0)),
        ],
        out_specs=[
            pl.BlockSpec((_BN, HID), lambda i: (i, 0)),
            pl.BlockSpec((_BN, ka), lambda i: (i, 0)),
        ],
        out_shape=[
            jax.ShapeDtypeStruct((N, HID), f32),
            jax.ShapeDtypeStruct((N, ka), f32),
        ],
    )(x, w, b, wa)


def _tsum_body(a0_ref, a1_ref, kw_ref, kb_ref, o_ref):
    @pl.when(pl.program_id(0) == 0)
    def _():
        o_ref[...] = jnp.zeros_like(o_ref)

    kw = kw_ref[...]
    kb = kb_ref[...]
    t0 = jnp.tanh(jnp.dot(jnp.maximum(a0_ref[...], 0.0), kw,
                          preferred_element_type=f32) + kb)
    t1 = jnp.tanh(jnp.dot(jnp.maximum(a1_ref[...], 0.0), kw,
                          preferred_element_type=f32) + kb)
    o_ref[0:1, :] += jnp.sum(t0, axis=0, keepdims=True)
    o_ref[1:2, :] += jnp.sum(t1, axis=0, keepdims=True)


def _tsum(a0, a1, kw, kb):
    return pl.pallas_call(
        _tsum_body,
        grid=(_GRID,),
        in_specs=[
            pl.BlockSpec((_BN, HID), lambda i: (i, 0)),
            pl.BlockSpec((_BN, HID), lambda i: (i, 0)),
            pl.BlockSpec((HID, HID), lambda i: (0, 0)),
            pl.BlockSpec((1, HID), lambda i: (0, 0)),
        ],
        out_specs=pl.BlockSpec((8, HID), lambda i: (0, 0)),
        out_shape=jax.ShapeDtypeStruct((8, HID), f32),
    )(a0, a1, kw, kb)


def _ln(m, g, b):
    mu = jnp.mean(m, axis=1, keepdims=True)
    d = m - mu
    var = jnp.mean(d * d, axis=1, keepdims=True)
    return d * lax.rsqrt(var + 1e-5) * g + b


def _merge2_body(a0_ref, a1_ref, t_ref, q_ref, g_ref, b_ref, o_ref):
    q = q_ref[...]
    s0 = jnp.sum(q * t_ref[0:1, :]) / N
    s1 = jnp.sum(q * t_ref[1:2, :]) / N
    mx = jnp.maximum(s0, s1)
    e0 = jnp.exp(s0 - mx)
    e1 = jnp.exp(s1 - mx)
    at0 = e0 / (e0 + e1)
    at1 = e1 / (e0 + e1)
    m = at0 * jnp.maximum(a0_ref[...], 0.0) + at1 * jnp.maximum(a1_ref[...], 0.0)
    o_ref[...] = _ln(m, g_ref[...], b_ref[...])


def _merge2(a0, a1, t, q, g, b):
    return pl.pallas_call(
        _merge2_body,
        grid=(_GRID,),
        in_specs=[
            pl.BlockSpec((_BN, HID), lambda i: (i, 0)),
            pl.BlockSpec((_BN, HID), lambda i: (i, 0)),
            pl.BlockSpec((8, HID), lambda i: (0, 0)),
            pl.BlockSpec((1, HID), lambda i: (0, 0)),
            pl.BlockSpec((1, HID), lambda i: (0, 0)),
            pl.BlockSpec((1, HID), lambda i: (0, 0)),
        ],
        out_specs=pl.BlockSpec((_BN, HID), lambda i: (i, 0)),
        out_shape=jax.ShapeDtypeStruct((N, HID), f32),
    )(a0, a1, t, q, g, b)


def _merge1_body(a0_ref, g_ref, b_ref, o_ref):
    o_ref[...] = _ln(jnp.maximum(a0_ref[...], 0.0), g_ref[...], b_ref[...])


def _merge1(a0, g, b):
    return pl.pallas_call(
        _merge1_body,
        grid=(_GRID,),
        in_specs=[
            pl.BlockSpec((_BN, HID), lambda i: (i, 0)),
            pl.BlockSpec((1, HID), lambda i: (0, 0)),
            pl.BlockSpec((1, HID), lambda i: (0, 0)),
        ],
        out_specs=pl.BlockSpec((_BN, HID), lambda i: (i, 0)),
        out_shape=jax.ShapeDtypeStruct((N, HID), f32),
    )(a0, g, b)


def _head_body(r0_ref, r1_ref, fw_ref, fpw_ref, fpb_ref, w1_ref, b1_ref,
               w2_ref, b2_ref, o_ref):
    wsum = fw_ref[0, 0] + fw_ref[0, 1]
    z = (fw_ref[0, 0] * r0_ref[...] + fw_ref[0, 1] * r1_ref[...]) / wsum
    z = jnp.maximum(jnp.dot(z, fpw_ref[...], preferred_element_type=f32)
                    + fpb_ref[...], 0.0)
    z = jnp.maximum(jnp.dot(z, w1_ref[...], preferred_element_type=f32)
                    + b1_ref[...], 0.0)
    o_ref[...] = jnp.dot(z, w2_ref[...], preferred_element_type=f32) + b2_ref[...]


def _head(r0, r1, fw, fpw, fpb, w1, b1, w2, b2):
    return pl.pallas_call(
        _head_body,
        grid=(_GRID,),
        in_specs=[
            pl.BlockSpec((_BN, HID), lambda i: (i, 0)),
            pl.BlockSpec((_BN, HID), lambda i: (i, 0)),
            pl.BlockSpec((1, HID), lambda i: (0, 0)),
            pl.BlockSpec((HID, HID), lambda i: (0, 0)),
            pl.BlockSpec((1, HID), lambda i: (0, 0)),
            pl.BlockSpec((HID, HID), lambda i: (0, 0)),
            pl.BlockSpec((1, HID), lambda i: (0, 0)),
            pl.BlockSpec((HID, OUT), lambda i: (0, 0)),
            pl.BlockSpec((1, OUT), lambda i: (0, 0)),
        ],
        out_specs=pl.BlockSpec((_BN, OUT), lambda i: (i, 0)),
        out_shape=jax.ShapeDtypeStruct((N, OUT), f32),
    )(r0, r1, fw, fpw, fpb, w1, b1, w2, b2)


def _band(lin):
    eye = jnp.eye(HEADS, 16, dtype=f32)
    w3 = jnp.einsum('hd,hk->hdk', lin, eye)
    return w3.reshape(HID, 16)


def _row128(*vals):
    v = jnp.stack([v.astype(f32) for v in vals])
    return jnp.pad(v, (0, 128 - v.shape[0])).reshape(1, 128)


def kernel(x_snp, x_gene, edge_sg, edge_gs, edge_gg, params):
    p = params
    edges = {'sg': edge_sg, 'gs': edge_gs, 'gg': edge_gg}
    xd = {'snp': x_snp, 'gene': x_gene}
    louts = []
    for i in range(2):
        wa_snp = jnp.concatenate(
            [_band(p['lin_src_sg_%d' % i]), _band(p['lin_dst_gs_%d' % i])],
            axis=1)
        wa_gene = jnp.concatenate(
            [_band(p['lin_dst_sg_%d' % i]), _band(p['lin_src_gs_%d' % i]),
             _band(p['lin_src_gg_%d' % i]), _band(p['lin_dst_gg_%d' % i])],
            axis=1)
        h_snp, a_snp = _proj(xd['snp'], p['proj_w_snp_%d' % i],
                             p['proj_b_snp_%d' % i].reshape(1, HID), wa_snp)
        h_gene, a_gene = _proj(xd['gene'], p['proj_w_gene_%d' % i],
                               p['proj_b_gene_%d' % i].reshape(1, HID), wa_gene)
        a_src_sg, a_dst_gs = a_snp[:, 0:16], a_snp[:, 16:32]
        a_dst_sg, a_src_gs = a_gene[:, 0:16], a_gene[:, 16:32]
        a_src_gg, a_dst_gg = a_gene[:, 32:48], a_gene[:, 48:64]

        agg_sg = _sc_messages(edges['sg'][0], edges['sg'][1],
                              a_src_sg, a_dst_sg, h_snp)
        agg_gs = _sc_messages(edges['gs'][0], edges['gs'][1],
                              a_src_gs, a_dst_gs, h_gene)
        agg_gg = _sc_messages(edges['gg'][0], edges['gg'][1],
                              a_src_gg, a_dst_gg, h_gene)

        kw = p['k_w_%d' % i]
        kb = p['k_b_%d' % i].reshape(1, HID)
        g = p['ln_g_%d' % i].reshape(1, HID)
        b = p['ln_b_%d' % i].reshape(1, HID)
        t = _tsum(agg_sg, agg_gg, kw, kb)
        res_gene = _merge2(agg_sg, agg_gg, t, p['q_%d' % i].reshape(1, HID),
                           g, b)
        res_snp = _merge1(agg_gs, g, b)
        xd = {'snp': res_snp, 'gene': res_gene}
        louts.append(xd)

    fw = _row128(p['fusion_w'][0], p['fusion_w'][1])
    args = (fw, p['fp_w'], p['fp_b'].reshape(1, HID),
            p['ow1'], p['ob1'].reshape(1, HID),
            p['ow2'], p['ob2'].reshape(1, OUT))
    o_snp = _head(louts[0]['snp'], louts[1]['snp'], *args)
    o_gene = _head(louts[0]['gene'], louts[1]['gene'], *args)
    return jnp.concatenate([o_snp, o_gene], axis=0)
